# Initial kernel scaffold; baseline (speedup 1.0000x reference)
import jax
import jax.numpy as jnp
from jax.experimental import pallas as pl


def kernel(x_patent, x_author, edge_index_pp, edge_index_ap, edge_index_pa, params):
    raise NotImplementedError("write your pallas kernel here")


# speedup vs baseline: 16.8371x; 16.8371x over previous
import functools

import jax
import jax.numpy as jnp
from jax import lax
from jax.experimental import pallas as pl
from jax.experimental.pallas import tpu as pltpu
from jax.experimental.pallas import tpu_sc as plsc

N = 10000
NP_ = 10112
D = 128
HEADS = 8
DH = 16
NCLS = 40
NC, NS = 2, 16
CH = 48
RPS = NP_ // NS
DUMMY = 10048
BS = 2528
GRID = NP_ // BS

E_RAW = 320000
E_SL = E_RAW + N
NCHS_PLAIN = 418
NCHS_GAT = 430
EPAD_PLAIN = NS * CH * NCHS_PLAIN
EPAD_SL = NS * CH * NCHS_GAT
f32 = jnp.float32


_MESH = plsc.VectorSubcoreMesh(core_axis_name="c", subcore_axis_name="s",
                               num_cores=NC, num_subcores=NS)


def _make_plain(nchunks):
    @functools.partial(
        pl.kernel,
        out_type=jax.ShapeDtypeStruct((NC, NP_, D), f32),
        mesh=_MESH,
        scratch_types=[
            pltpu.VMEM_SHARED((NP_, D), f32),
            pltpu.VMEM((CH,), jnp.int32),
            pltpu.VMEM((CH,), jnp.int32),
            pltpu.VMEM((CH, D), f32),
            pltpu.VMEM((CH, D), f32),
            pltpu.SemaphoreType.DMA,
        ],
    )
    def plain_pass(table, srci, dsti, zeros128, ones128,
                   out128,
                   acc128, idx_s, idx_d, rows, ones_v, sem):
        c = lax.axis_index("c")
        s = lax.axis_index("s")
        r0 = s * RPS
        pltpu.sync_copy(zeros128, acc128.at[pl.ds(r0, RPS)])
        pltpu.sync_copy(ones128, ones_v)
        plsc.subcore_barrier()

        base0 = s * (nchunks * CH)

        @pl.loop(0, nchunks)
        def _(i):
            b = base0 + i * CH
            pltpu.sync_copy(dsti.at[pl.ds(b, CH)], idx_d)

            @pl.when(c == 0)
            def _():
                pltpu.sync_copy(srci.at[pl.ds(b, CH)], idx_s)
                pltpu.async_copy(table.at[idx_s], rows, sem).wait()
                pltpu.sync_copy(rows, acc128.at[idx_d], add=True)

            @pl.when(c == 1)
            def _():
                pltpu.sync_copy(ones_v, acc128.at[idx_d], add=True)

        plsc.subcore_barrier()
        pltpu.sync_copy(acc128.at[pl.ds(r0, RPS)], out128.at[c, pl.ds(r0, RPS)])

    return plain_pass


def _make_gat(nchunks):
    @functools.partial(
        pl.kernel,
        out_type=jax.ShapeDtypeStruct((NC, NP_, D), f32),
        mesh=_MESH,
        scratch_types=[
            pltpu.VMEM_SHARED((NP_, D), f32),
            pltpu.VMEM((CH,), jnp.int32),
            pltpu.VMEM((CH,), jnp.int32),
            pltpu.VMEM((CH, D), f32),
            pltpu.VMEM((CH, D), f32),
            pltpu.VMEM((CH, D), f32),
            pltpu.VMEM((CH, D), f32),
            pltpu.SemaphoreType.DMA,
        ],
    )
    def gat_pass(htab, acat, bcat, srci, dsti, zeros128,
                 out128,
                 acc128, idx_s, idx_d, hbuf, abuf, bbuf, wbuf, sem):
        c = lax.axis_index("c")
        s = lax.axis_index("s")
        r0 = s * RPS
        pltpu.sync_copy(zeros128, acc128.at[pl.ds(r0, RPS)])

        zero16 = jnp.zeros((16,), f32)

        @pl.when(c == 1)
        def _():
            @pl.loop(0, CH)
            def _(e):
                for k in range(1, HEADS):
                    wbuf[e, pl.ds(k * DH, DH)] = zero16

        plsc.subcore_barrier()

        base0 = s * (nchunks * CH)
        lanes = lax.iota(jnp.int32, 16)

        @pl.loop(0, nchunks)
        def _(i):
            b = base0 + i * CH
            pltpu.sync_copy(srci.at[pl.ds(b, CH)], idx_s)
            pltpu.sync_copy(dsti.at[pl.ds(b, CH)], idx_d)
            pltpu.async_copy(acat.at[idx_s], abuf, sem).wait()
            pltpu.async_copy(bcat.at[idx_d], bbuf, sem).wait()

            @pl.when(c == 0)
            def _():
                pltpu.async_copy(htab.at[idx_s], hbuf, sem).wait()

                @pl.loop(0, CH)
                def _(e):
                    v = abuf[e, pl.ds(0, 16)] + bbuf[e, pl.ds(0, 16)]
                    vc = jnp.where(lanes < 8,
                                   jnp.where(v >= 0.0, v, 0.2 * v), 0.0)
                    t = jnp.exp(vc)
                    for j in range(HEADS):
                        hv = hbuf[e, pl.ds(j * DH, DH)]
                        wbuf[e, pl.ds(j * DH, DH)] = hv * t[j]

            @pl.when(c == 1)
            def _():
                @pl.loop(0, CH)
                def _(e):
                    v = abuf[e, pl.ds(0, 16)] + bbuf[e, pl.ds(0, 16)]
                    vc = jnp.where(lanes < 8,
                                   jnp.where(v >= 0.0, v, 0.2 * v), 0.0)
                    wbuf[e, pl.ds(0, 16)] = jnp.exp(vc)

            pltpu.sync_copy(wbuf, acc128.at[idx_d], add=True)

        plsc.subcore_barrier()
        pltpu.sync_copy(acc128.at[pl.ds(r0, RPS)], out128.at[c, pl.ds(r0, RPS)])

    return gat_pass


_PLAIN_P = _make_plain(NCHS_PLAIN)
_PLAIN_G = _make_plain(NCHS_GAT)
_GAT_G = _make_gat(NCHS_GAT)


def _lnk(x, g, b, eps=1e-5):
    m = jnp.mean(x, axis=-1, keepdims=True)
    v = jnp.mean((x - m) * (x - m), axis=-1, keepdims=True)
    return (x - m) * lax.rsqrt(v + eps) * g + b


def _geluk(x):
    return 0.5 * x * (1.0 + lax.erf(x * 0.7071067811865476))


def _dot(a, b):
    return jnp.dot(a, b, preferred_element_type=f32)


def _nspec(a):
    if a.ndim == 2 and a.shape[0] == NP_:
        w = a.shape[1]
        return pl.BlockSpec((BS, w), lambda i: (i, 0))
    if a.ndim == 3 and a.shape[1] == NP_:
        d0, _, w = a.shape
        return pl.BlockSpec((d0, BS, w), lambda i: (0, i, 0))
    nd = a.ndim
    return pl.BlockSpec(a.shape, lambda i: (0,) * nd)


def _tc_call(body, args, out_widths):
    outs = tuple(jax.ShapeDtypeStruct((NP_, w), f32) for w in out_widths)
    return pl.pallas_call(
        body,
        grid=(GRID,),
        in_specs=[_nspec(a) for a in args],
        out_specs=tuple(pl.BlockSpec((BS, w), lambda i: (i, 0)) for w in out_widths),
        out_shape=outs,
    )(*args)


def _t1_body(xp_r, xa_r, png, pnb, ang, anb,
             plW1, plb1, plg, plbln, plW2, plb2,
             alW1, alb1, alg, albln, alW2, alb2,
             g1W, g1as, g1ad,
             xp0_o, xa0_o, h1_o, acat_o, bcat_o):
    xp = _lnk(xp_r[...], png[...], pnb[...])
    xa = _lnk(xa_r[...], ang[...], anb[...])
    t = _geluk(_dot(xp, plW1[...]) + plb1[...])
    t = _lnk(t, plg[...], plbln[...])
    xp0 = _dot(t, plW2[...]) + plb2[...]
    t = _geluk(_dot(xa, alW1[...]) + alb1[...])
    t = _lnk(t, alg[...], albln[...])
    xa0 = _dot(t, alW2[...]) + alb2[...]
    h1 = _dot(xp0, g1W[...])
    hh = h1.reshape(BS, HEADS, DH)
    a_s = jnp.sum(hh * g1as[...], axis=-1)
    a_d = jnp.sum(hh * g1ad[...], axis=-1)
    xp0_o[...] = xp0
    xa0_o[...] = xa0
    h1_o[...] = h1
    acat_o[...] = jnp.concatenate([a_s, a_d], axis=-1)
    bcat_o[...] = jnp.concatenate([a_d, a_s], axis=-1)


def _t2_body(xp0_r, xa0_r, po, pa, ps, pc,
             g1b, n1g, n1b, sWl, sbl, sWr, n2g, n2b, gcnW,
             xp1_o, xp2_o, hs_o):
    o = po[...]
    pa_v = pa[...]
    asum = pa_v[:, :HEADS]
    att = (o.reshape(BS, HEADS, DH) / (asum[:, :, None] + 1e-16)).reshape(BS, D)
    att1 = _geluk(_lnk(att + g1b[...], n1g[...], n1b[...]))
    xp1 = att1 + xp0_r[...]
    ssum = ps[...]
    cnt = pc[...][:, 0:1]
    mean = ssum / jnp.maximum(cnt, 1.0)
    sage = _dot(mean, sWl[...]) + sbl[...] + _dot(xa0_r[...], sWr[...])
    a2p = _geluk(_lnk(sage, n2g[...], n2b[...]))
    xp2 = xp1 + 0.5 * a2p
    deg = pa_v[:, HEADS:HEADS + 1]
    dinv = jnp.where(deg > 0.0, lax.rsqrt(jnp.maximum(deg, 1e-30)), 0.0)
    hs = _dot(xp2, gcnW[...]) * dinv
    xp1_o[...] = xp1
    xp2_o[...] = xp2
    hs_o[...] = hs


def _t3_body(xp2_r, pg, pa, gcnb, g2W, g2as, g2ad,
             xp3_o, h2_o, acat_o, bcat_o):
    g = pg[...]
    deg = pa[...][:, HEADS:HEADS + 1]
    dinv = jnp.where(deg > 0.0, lax.rsqrt(jnp.maximum(deg, 1e-30)), 0.0)
    conv = _geluk(g * dinv + gcnb[...])
    xp3 = xp2_r[...] + 0.3 * conv
    h2 = _dot(xp3, g2W[...])
    hh = h2.reshape(BS, HEADS, DH)
    a_s = jnp.sum(hh * g2as[...], axis=-1)
    a_d = jnp.sum(hh * g2ad[...], axis=-1)
    xp3_o[...] = xp3
    h2_o[...] = h2
    acat_o[...] = jnp.concatenate([a_s, a_d], axis=-1)
    bcat_o[...] = jnp.concatenate([a_d, a_s], axis=-1)


def _t4_body(xp0_r, xp1_r, xp3_r, po, pa, g2b, n4g, n4b, sw,
             cW1, cb1, cg, cbln, cW2, cb2, cW3, cb3,
             out_o):
    o = po[...]
    asum = pa[...][:, :HEADS]
    att = (o.reshape(BS, HEADS, DH) / (asum[:, :, None] + 1e-16)).reshape(BS, D)
    att2 = _geluk(_lnk(att + g2b[...], n4g[...], n4b[...]))
    xp4 = xp3_r[...] + att2
    swv = sw[...]
    ms = swv[0, 0] * xp0_r[...] + swv[0, 1] * xp1_r[...] + swv[0, 2] * xp4
    comb = jnp.concatenate([ms, xp4], axis=-1)
    h1c = _geluk(_lnk(_dot(comb, cW1[...]) + cb1[...], cg[...], cbln[...]))
    h2c = _geluk(_dot(h1c, cW2[...]) + cb2[...])
    out_o[...] = _dot(h2c, cW3[...]) + cb3[...]


def _pad_edges(src, dst, epad):
    npad = epad - src.shape[0]
    src = jnp.concatenate([src, jnp.zeros((npad,), jnp.int32)])
    dst = jnp.concatenate([dst, jnp.full((npad,), DUMMY, jnp.int32)])
    return src, dst


def kernel(x_patent, x_author, edge_index_pp, edge_index_ap, edge_index_pa,
           params):
    P = params
    r1 = lambda p: p.reshape(1, -1)

    xp_in = jnp.pad(x_patent, ((0, NP_ - N), (0, 0)))
    xa_in = jnp.pad(x_author, ((0, NP_ - N), (0, 0)))

    loops = jnp.arange(N, dtype=jnp.int32)
    pp_src = jnp.concatenate([edge_index_pp[0], loops])
    pp_dst = jnp.concatenate([edge_index_pp[1], loops])
    pp_src, pp_dst = _pad_edges(pp_src, pp_dst, EPAD_SL)
    ap_src, ap_dst = _pad_edges(edge_index_ap[0], edge_index_ap[1], EPAD_PLAIN)

    zeros128 = jnp.zeros((RPS, D), f32)
    ones128 = jnp.ones((CH, D), f32)
    padw = ((0, 0), (0, D - 16))

    sw = jax.nn.softmax(P['scale_w'])
    sw_arr = jnp.zeros((1, D), f32).at[0, :3].set(sw)
    cW3 = jnp.pad(P['c_W3'], ((0, 0), (0, D - NCLS)))
    cb3 = jnp.pad(P['c_b3'], ((0, D - NCLS))).reshape(1, D)

    xp0, xa0, h1, acat1, bcat1 = _tc_call(
        _t1_body,
        [xp_in, xa_in, r1(P['pn_g']), r1(P['pn_b']), r1(P['an_g']), r1(P['an_b']),
         P['pl_W1'], r1(P['pl_b1']), r1(P['pl_g']), r1(P['pl_bln']), P['pl_W2'], r1(P['pl_b2']),
         P['al_W1'], r1(P['al_b1']), r1(P['al_g']), r1(P['al_bln']), P['al_W2'], r1(P['al_b2']),
         P['g1_W'], P['g1_as'], P['g1_ad']],
        (D, D, D, 16, 16))

    g1 = _GAT_G(h1, jnp.pad(acat1, padw), jnp.pad(bcat1, padw),
                pp_src, pp_dst, zeros128)
    g1_out, g1_aux = g1[0], g1[1]
    sage = _PLAIN_P(xa0, ap_src, ap_dst, zeros128, ones128)
    sage_sum, sage_cnt = sage[0], sage[1]

    xp1, xp2, hs = _tc_call(
        _t2_body,
        [xp0, xa0, g1_out, g1_aux, sage_sum, sage_cnt,
         r1(P['g1_b']), r1(P['n1_g']), r1(P['n1_b']),
         P['sap_Wl'], r1(P['sap_bl']), P['sap_Wr'],
         r1(P['n2_g']), r1(P['n2_b']), P['gcn_W']],
        (D, D, D))

    gcn = _PLAIN_G(hs, pp_src, pp_dst, zeros128, ones128)

    xp3, h2, acat2, bcat2 = _tc_call(
        _t3_body,
        [xp2, gcn[0], g1_aux, r1(P['gcn_b']), P['g2_W'], P['g2_as'], P['g2_ad']],
        (D, D, 16, 16))

    g2 = _GAT_G(h2, jnp.pad(acat2, padw), jnp.pad(bcat2, padw),
                pp_src, pp_dst, zeros128)

    out_pad, = _tc_call(
        _t4_body,
        [xp0, xp1, xp3, g2[0], g2[1],
         r1(P['g2_b']), r1(P['n4_g']), r1(P['n4_b']), sw_arr,
         P['c_W1'], r1(P['c_b1']), r1(P['c_g']), r1(P['c_bln']),
         P['c_W2'], r1(P['c_b2']), cW3, cb3],
        (D,))

    return out_pad[:N, :NCLS]


# speedup vs baseline: 25.7128x; 1.5272x over previous
import functools

import jax
import jax.numpy as jnp
from jax import lax
from jax.experimental import pallas as pl
from jax.experimental.pallas import tpu as pltpu
from jax.experimental.pallas import tpu_sc as plsc

N = 10000
NP_ = 10112
D = 128
HEADS = 8
DH = 16
NCLS = 40
NC, NS = 2, 16
CH = 72
RPS = NP_ // NS
DUMMY = 10048
BS = 2528
GRID = NP_ // BS

E_RAW = 320000
E_SL = E_RAW + N
NCHS_PLAIN = 278
NCHS_GAT = 287
EPAD_PLAIN = NS * CH * NCHS_PLAIN
EPAD_SL = NS * CH * NCHS_GAT
f32 = jnp.float32


_MESH = plsc.VectorSubcoreMesh(core_axis_name="c", subcore_axis_name="s",
                               num_cores=NC, num_subcores=NS)


def _make_plain(nchunks):
    @functools.partial(
        pl.kernel,
        out_type=jax.ShapeDtypeStruct((NC, NP_, D), f32),
        mesh=_MESH,
        scratch_types=[
            pltpu.VMEM_SHARED((NP_, D), f32),
            pltpu.VMEM((CH,), jnp.int32),
            pltpu.VMEM((CH,), jnp.int32),
            pltpu.VMEM((CH, D), f32),
            pltpu.VMEM((CH, D), f32),
            pltpu.SemaphoreType.DMA,
        ],
    )
    def plain_pass(table, srci, dsti, zeros128, ones128,
                   out128,
                   acc128, idx_s, idx_d, rows, ones_v, sem):
        c = lax.axis_index("c")
        s = lax.axis_index("s")
        r0 = s * RPS
        pltpu.sync_copy(zeros128, acc128.at[pl.ds(r0, RPS)])
        pltpu.sync_copy(ones128, ones_v)
        plsc.subcore_barrier()

        base0 = s * (nchunks * CH)

        @pl.loop(0, nchunks)
        def _(i):
            b = base0 + i * CH
            pltpu.sync_copy(dsti.at[pl.ds(b, CH)], idx_d)

            @pl.when(c == 0)
            def _():
                pltpu.sync_copy(srci.at[pl.ds(b, CH)], idx_s)
                pltpu.async_copy(table.at[idx_s], rows, sem).wait()
                pltpu.sync_copy(rows, acc128.at[idx_d], add=True)

            @pl.when(c == 1)
            def _():
                pltpu.sync_copy(ones_v, acc128.at[idx_d], add=True)

        plsc.subcore_barrier()
        pltpu.sync_copy(acc128.at[pl.ds(r0, RPS)], out128.at[c, pl.ds(r0, RPS)])

    return plain_pass


def _make_gat(nchunks):
    @functools.partial(
        pl.kernel,
        out_type=jax.ShapeDtypeStruct((NC, NP_, D), f32),
        mesh=_MESH,
        scratch_types=[
            pltpu.VMEM_SHARED((NP_, D), f32),
            pltpu.VMEM((CH,), jnp.int32),
            pltpu.VMEM((CH,), jnp.int32),
            pltpu.VMEM((CH, D), f32),
            pltpu.VMEM((CH, D), f32),
            pltpu.VMEM((CH, D), f32),
            pltpu.VMEM((CH, D), f32),
            pltpu.SemaphoreType.DMA,
        ],
    )
    def gat_pass(htab, acat, bcat, srci, dsti, zeros128,
                 out128,
                 acc128, idx_s, idx_d, hbuf, abuf, bbuf, wbuf, sem):
        c = lax.axis_index("c")
        s = lax.axis_index("s")
        r0 = s * RPS
        pltpu.sync_copy(zeros128, acc128.at[pl.ds(r0, RPS)])

        zero16 = jnp.zeros((16,), f32)

        @pl.when(c == 1)
        def _():
            @pl.loop(0, CH)
            def _(e):
                for k in range(1, HEADS):
                    wbuf[e, pl.ds(k * DH, DH)] = zero16

        plsc.subcore_barrier()

        base0 = s * (nchunks * CH)
        lanes = lax.iota(jnp.int32, 16)

        @pl.loop(0, nchunks)
        def _(i):
            b = base0 + i * CH
            pltpu.sync_copy(srci.at[pl.ds(b, CH)], idx_s)
            pltpu.sync_copy(dsti.at[pl.ds(b, CH)], idx_d)

            @pl.when(c == 0)
            def _():
                ca = pltpu.async_copy(acat.at[idx_s], abuf, sem)
                cb = pltpu.async_copy(bcat.at[idx_d], bbuf, sem)
                chh = pltpu.async_copy(htab.at[idx_s], hbuf, sem)
                ca.wait()
                cb.wait()
                chh.wait()

                @pl.loop(0, CH)
                def _(e):
                    v = abuf[e, pl.ds(0, 16)] + bbuf[e, pl.ds(0, 16)]
                    vc = jnp.where(lanes < 8,
                                   jnp.where(v >= 0.0, v, 0.2 * v), 0.0)
                    t = jnp.exp(vc)
                    for j in range(HEADS):
                        hv = hbuf[e, pl.ds(j * DH, DH)]
                        wbuf[e, pl.ds(j * DH, DH)] = hv * t[j]

            @pl.when(c == 1)
            def _():
                ca = pltpu.async_copy(acat.at[idx_s], abuf, sem)
                cb = pltpu.async_copy(bcat.at[idx_d], bbuf, sem)
                ca.wait()
                cb.wait()

                @pl.loop(0, CH)
                def _(e):
                    v = abuf[e, pl.ds(0, 16)] + bbuf[e, pl.ds(0, 16)]
                    vc = jnp.where(lanes < 8,
                                   jnp.where(v >= 0.0, v, 0.2 * v), 0.0)
                    wbuf[e, pl.ds(0, 16)] = jnp.exp(vc)

            pltpu.sync_copy(wbuf, acc128.at[idx_d], add=True)

        plsc.subcore_barrier()
        pltpu.sync_copy(acc128.at[pl.ds(r0, RPS)], out128.at[c, pl.ds(r0, RPS)])

    return gat_pass


_PLAIN_P = _make_plain(NCHS_PLAIN)
_PLAIN_G = _make_plain(NCHS_GAT)
_GAT_G = _make_gat(NCHS_GAT)


def _lnk(x, g, b, eps=1e-5):
    m = jnp.mean(x, axis=-1, keepdims=True)
    v = jnp.mean((x - m) * (x - m), axis=-1, keepdims=True)
    return (x - m) * lax.rsqrt(v + eps) * g + b


def _geluk(x):
    return 0.5 * x * (1.0 + lax.erf(x * 0.7071067811865476))


def _dot(a, b):
    return jnp.dot(a, b, preferred_element_type=f32)


def _nspec(a):
    if a.ndim == 2 and a.shape[0] == NP_:
        w = a.shape[1]
        return pl.BlockSpec((BS, w), lambda i: (i, 0))
    if a.ndim == 3 and a.shape[1] == NP_:
        d0, _, w = a.shape
        return pl.BlockSpec((d0, BS, w), lambda i: (0, i, 0))
    nd = a.ndim
    return pl.BlockSpec(a.shape, lambda i: (0,) * nd)


def _tc_call(body, args, out_widths):
    outs = tuple(jax.ShapeDtypeStruct((NP_, w), f32) for w in out_widths)
    return pl.pallas_call(
        body,
        grid=(GRID,),
        in_specs=[_nspec(a) for a in args],
        out_specs=tuple(pl.BlockSpec((BS, w), lambda i: (i, 0)) for w in out_widths),
        out_shape=outs,
    )(*args)


def _t1_body(xp_r, xa_r, png, pnb, ang, anb,
             plW1, plb1, plg, plbln, plW2, plb2,
             alW1, alb1, alg, albln, alW2, alb2,
             g1W, g1as, g1ad,
             xp0_o, xa0_o, h1_o, acat_o, bcat_o):
    xp = _lnk(xp_r[...], png[...], pnb[...])
    xa = _lnk(xa_r[...], ang[...], anb[...])
    t = _geluk(_dot(xp, plW1[...]) + plb1[...])
    t = _lnk(t, plg[...], plbln[...])
    xp0 = _dot(t, plW2[...]) + plb2[...]
    t = _geluk(_dot(xa, alW1[...]) + alb1[...])
    t = _lnk(t, alg[...], albln[...])
    xa0 = _dot(t, alW2[...]) + alb2[...]
    h1 = _dot(xp0, g1W[...])
    hh = h1.reshape(BS, HEADS, DH)
    a_s = jnp.sum(hh * g1as[...], axis=-1)
    a_d = jnp.sum(hh * g1ad[...], axis=-1)
    xp0_o[...] = xp0
    xa0_o[...] = xa0
    h1_o[...] = h1
    acat_o[...] = jnp.concatenate([a_s, a_d], axis=-1)
    bcat_o[...] = jnp.concatenate([a_d, a_s], axis=-1)


def _t2_body(xp0_r, xa0_r, po, pa, ps, pc,
             g1b, n1g, n1b, sWl, sbl, sWr, n2g, n2b, gcnW,
             xp1_o, xp2_o, hs_o):
    o = po[...]
    pa_v = pa[...]
    asum = pa_v[:, :HEADS]
    att = (o.reshape(BS, HEADS, DH) / (asum[:, :, None] + 1e-16)).reshape(BS, D)
    att1 = _geluk(_lnk(att + g1b[...], n1g[...], n1b[...]))
    xp1 = att1 + xp0_r[...]
    ssum = ps[...]
    cnt = pc[...][:, 0:1]
    mean = ssum / jnp.maximum(cnt, 1.0)
    sage = _dot(mean, sWl[...]) + sbl[...] + _dot(xa0_r[...], sWr[...])
    a2p = _geluk(_lnk(sage, n2g[...], n2b[...]))
    xp2 = xp1 + 0.5 * a2p
    deg = pa_v[:, HEADS:HEADS + 1]
    dinv = jnp.where(deg > 0.0, lax.rsqrt(jnp.maximum(deg, 1e-30)), 0.0)
    hs = _dot(xp2, gcnW[...]) * dinv
    xp1_o[...] = xp1
    xp2_o[...] = xp2
    hs_o[...] = hs


def _t3_body(xp2_r, pg, pa, gcnb, g2W, g2as, g2ad,
             xp3_o, h2_o, acat_o, bcat_o):
    g = pg[...]
    deg = pa[...][:, HEADS:HEADS + 1]
    dinv = jnp.where(deg > 0.0, lax.rsqrt(jnp.maximum(deg, 1e-30)), 0.0)
    conv = _geluk(g * dinv + gcnb[...])
    xp3 = xp2_r[...] + 0.3 * conv
    h2 = _dot(xp3, g2W[...])
    hh = h2.reshape(BS, HEADS, DH)
    a_s = jnp.sum(hh * g2as[...], axis=-1)
    a_d = jnp.sum(hh * g2ad[...], axis=-1)
    xp3_o[...] = xp3
    h2_o[...] = h2
    acat_o[...] = jnp.concatenate([a_s, a_d], axis=-1)
    bcat_o[...] = jnp.concatenate([a_d, a_s], axis=-1)


def _t4_body(xp0_r, xp1_r, xp3_r, po, pa, g2b, n4g, n4b, sw,
             cW1, cb1, cg, cbln, cW2, cb2, cW3, cb3,
             out_o):
    o = po[...]
    asum = pa[...][:, :HEADS]
    att = (o.reshape(BS, HEADS, DH) / (asum[:, :, None] + 1e-16)).reshape(BS, D)
    att2 = _geluk(_lnk(att + g2b[...], n4g[...], n4b[...]))
    xp4 = xp3_r[...] + att2
    swv = sw[...]
    ms = swv[0, 0] * xp0_r[...] + swv[0, 1] * xp1_r[...] + swv[0, 2] * xp4
    comb = jnp.concatenate([ms, xp4], axis=-1)
    h1c = _geluk(_lnk(_dot(comb, cW1[...]) + cb1[...], cg[...], cbln[...]))
    h2c = _geluk(_dot(h1c, cW2[...]) + cb2[...])
    out_o[...] = _dot(h2c, cW3[...]) + cb3[...]


def _pad_edges(src, dst, epad):
    npad = epad - src.shape[0]
    src = jnp.concatenate([src, jnp.zeros((npad,), jnp.int32)])
    dst = jnp.concatenate([dst, jnp.full((npad,), DUMMY, jnp.int32)])
    return src, dst


def kernel(x_patent, x_author, edge_index_pp, edge_index_ap, edge_index_pa,
           params):
    P = params
    r1 = lambda p: p.reshape(1, -1)

    xp_in = jnp.pad(x_patent, ((0, NP_ - N), (0, 0)))
    xa_in = jnp.pad(x_author, ((0, NP_ - N), (0, 0)))

    loops = jnp.arange(N, dtype=jnp.int32)
    pp_src = jnp.concatenate([edge_index_pp[0], loops])
    pp_dst = jnp.concatenate([edge_index_pp[1], loops])
    pp_src, pp_dst = _pad_edges(pp_src, pp_dst, EPAD_SL)
    ap_src, ap_dst = _pad_edges(edge_index_ap[0], edge_index_ap[1], EPAD_PLAIN)

    zeros128 = jnp.zeros((RPS, D), f32)
    ones128 = jnp.ones((CH, D), f32)
    padw = ((0, 0), (0, D - 16))

    sw = jax.nn.softmax(P['scale_w'])
    sw_arr = jnp.zeros((1, D), f32).at[0, :3].set(sw)
    cW3 = jnp.pad(P['c_W3'], ((0, 0), (0, D - NCLS)))
    cb3 = jnp.pad(P['c_b3'], ((0, D - NCLS))).reshape(1, D)

    xp0, xa0, h1, acat1, bcat1 = _tc_call(
        _t1_body,
        [xp_in, xa_in, r1(P['pn_g']), r1(P['pn_b']), r1(P['an_g']), r1(P['an_b']),
         P['pl_W1'], r1(P['pl_b1']), r1(P['pl_g']), r1(P['pl_bln']), P['pl_W2'], r1(P['pl_b2']),
         P['al_W1'], r1(P['al_b1']), r1(P['al_g']), r1(P['al_bln']), P['al_W2'], r1(P['al_b2']),
         P['g1_W'], P['g1_as'], P['g1_ad']],
        (D, D, D, 16, 16))

    g1 = _GAT_G(h1, jnp.pad(acat1, padw), jnp.pad(bcat1, padw),
                pp_src, pp_dst, zeros128)
    g1_out, g1_aux = g1[0], g1[1]
    sage = _PLAIN_P(xa0, ap_src, ap_dst, zeros128, ones128)
    sage_sum, sage_cnt = sage[0], sage[1]

    xp1, xp2, hs = _tc_call(
        _t2_body,
        [xp0, xa0, g1_out, g1_aux, sage_sum, sage_cnt,
         r1(P['g1_b']), r1(P['n1_g']), r1(P['n1_b']),
         P['sap_Wl'], r1(P['sap_bl']), P['sap_Wr'],
         r1(P['n2_g']), r1(P['n2_b']), P['gcn_W']],
        (D, D, D))

    gcn = _PLAIN_G(hs, pp_src, pp_dst, zeros128, ones128)

    xp3, h2, acat2, bcat2 = _tc_call(
        _t3_body,
        [xp2, gcn[0], g1_aux, r1(P['gcn_b']), P['g2_W'], P['g2_as'], P['g2_ad']],
        (D, D, 16, 16))

    g2 = _GAT_G(h2, jnp.pad(acat2, padw), jnp.pad(bcat2, padw),
                pp_src, pp_dst, zeros128)

    out_pad, = _tc_call(
        _t4_body,
        [xp0, xp1, xp3, g2[0], g2[1],
         r1(P['g2_b']), r1(P['n4_g']), r1(P['n4_b']), sw_arr,
         P['c_W1'], r1(P['c_b1']), r1(P['c_g']), r1(P['c_bln']),
         P['c_W2'], r1(P['c_b2']), cW3, cb3],
        (D,))

    return out_pad[:N, :NCLS]


# speedup vs baseline: 27.1327x; 1.0552x over previous
import functools

import jax
import jax.numpy as jnp
from jax import lax
from jax.experimental import pallas as pl
from jax.experimental.pallas import tpu as pltpu
from jax.experimental.pallas import tpu_sc as plsc

N = 10000
NP_ = 10112
D = 128
HEADS = 8
DH = 16
NCLS = 40
NC, NS = 2, 16
CH = 80
RPS = NP_ // NS
DUMMY = 10048
BS = 2528
GRID = NP_ // BS

E_RAW = 320000
E_SL = E_RAW + N
NCHS_PLAIN = 250
NCHS_GAT = 258
EPAD_PLAIN = NS * CH * NCHS_PLAIN
EPAD_SL = NS * CH * NCHS_GAT
f32 = jnp.float32


_MESH = plsc.VectorSubcoreMesh(core_axis_name="c", subcore_axis_name="s",
                               num_cores=NC, num_subcores=NS)


def _make_plain(nchunks):
    @functools.partial(
        pl.kernel,
        out_type=jax.ShapeDtypeStruct((NC, NP_, D), f32),
        mesh=_MESH,
        scratch_types=[
            pltpu.VMEM_SHARED((NP_, D), f32),
            pltpu.VMEM((CH,), jnp.int32),
            pltpu.VMEM((CH,), jnp.int32),
            pltpu.VMEM((CH, D), f32),
            pltpu.VMEM((CH, D), f32),
            pltpu.SemaphoreType.DMA,
        ],
    )
    def plain_pass(table, srci, dsti, zeros128, ones128,
                   out128,
                   acc128, idx_s, idx_d, rows, ones_v, sem):
        c = lax.axis_index("c")
        s = lax.axis_index("s")
        r0 = s * RPS
        pltpu.sync_copy(zeros128, acc128.at[pl.ds(r0, RPS)])
        pltpu.sync_copy(ones128, ones_v)
        plsc.subcore_barrier()

        base0 = s * (nchunks * CH)

        @pl.loop(0, nchunks)
        def _(i):
            b = base0 + i * CH
            pltpu.sync_copy(dsti.at[pl.ds(b, CH)], idx_d)

            @pl.when(c == 0)
            def _():
                pltpu.sync_copy(srci.at[pl.ds(b, CH)], idx_s)
                pltpu.async_copy(table.at[idx_s], rows, sem).wait()
                pltpu.sync_copy(rows, acc128.at[idx_d], add=True)

            @pl.when(c == 1)
            def _():
                pltpu.sync_copy(ones_v, acc128.at[idx_d], add=True)

        plsc.subcore_barrier()
        pltpu.sync_copy(acc128.at[pl.ds(r0, RPS)], out128.at[c, pl.ds(r0, RPS)])

    return plain_pass


def _make_gat(nchunks):
    @functools.partial(
        pl.kernel,
        out_type=jax.ShapeDtypeStruct((NC, NP_, D), f32),
        mesh=_MESH,
        scratch_types=[
            pltpu.VMEM_SHARED((NP_, D), f32),
            pltpu.VMEM((CH,), jnp.int32),
            pltpu.VMEM((CH,), jnp.int32),
            pltpu.VMEM((CH, D), f32),
            pltpu.VMEM((CH, D), f32),
            pltpu.VMEM((CH, D), f32),
            pltpu.VMEM((CH, D), f32),
            pltpu.SemaphoreType.DMA,
        ],
    )
    def gat_pass(htab, acat, bcat, srci, dsti, zeros128,
                 out128,
                 acc128, idx_s, idx_d, hbuf, abuf, bbuf, wbuf, sem):
        c = lax.axis_index("c")
        s = lax.axis_index("s")
        r0 = s * RPS
        pltpu.sync_copy(zeros128, acc128.at[pl.ds(r0, RPS)])

        zero16 = jnp.zeros((16,), f32)

        @pl.when(c == 1)
        def _():
            @pl.loop(0, CH)
            def _(e):
                for k in range(1, HEADS):
                    wbuf[e, pl.ds(k * DH, DH)] = zero16

        plsc.subcore_barrier()

        base0 = s * (nchunks * CH)
        lanes = lax.iota(jnp.int32, 16)

        @pl.loop(0, nchunks)
        def _(i):
            b = base0 + i * CH
            pltpu.sync_copy(srci.at[pl.ds(b, CH)], idx_s)
            pltpu.sync_copy(dsti.at[pl.ds(b, CH)], idx_d)

            @pl.when(c == 0)
            def _():
                ca = pltpu.async_copy(acat.at[idx_s], abuf, sem)
                cb = pltpu.async_copy(bcat.at[idx_d], bbuf, sem)
                chh = pltpu.async_copy(htab.at[idx_s], hbuf, sem)
                ca.wait()
                cb.wait()
                chh.wait()

                @pl.loop(0, CH)
                def _(e):
                    v = abuf[e, pl.ds(0, 16)] + bbuf[e, pl.ds(0, 16)]
                    vc = jnp.where(lanes < 8,
                                   jnp.where(v >= 0.0, v, 0.2 * v), 0.0)
                    t = jnp.exp(vc)
                    for j in range(HEADS):
                        hv = hbuf[e, pl.ds(j * DH, DH)]
                        wbuf[e, pl.ds(j * DH, DH)] = hv * t[j]

            @pl.when(c == 1)
            def _():
                ca = pltpu.async_copy(acat.at[idx_s], abuf, sem)
                cb = pltpu.async_copy(bcat.at[idx_d], bbuf, sem)
                ca.wait()
                cb.wait()

                @pl.loop(0, CH)
                def _(e):
                    v = abuf[e, pl.ds(0, 16)] + bbuf[e, pl.ds(0, 16)]
                    vc = jnp.where(lanes < 8,
                                   jnp.where(v >= 0.0, v, 0.2 * v), 0.0)
                    wbuf[e, pl.ds(0, 16)] = jnp.exp(vc)

            pltpu.sync_copy(wbuf, acc128.at[idx_d], add=True)

        plsc.subcore_barrier()
        pltpu.sync_copy(acc128.at[pl.ds(r0, RPS)], out128.at[c, pl.ds(r0, RPS)])

    return gat_pass


_PLAIN_P = _make_plain(NCHS_PLAIN)
_PLAIN_G = _make_plain(NCHS_GAT)
_GAT_G = _make_gat(NCHS_GAT)


def _lnk(x, g, b, eps=1e-5):
    m = jnp.mean(x, axis=-1, keepdims=True)
    v = jnp.mean((x - m) * (x - m), axis=-1, keepdims=True)
    return (x - m) * lax.rsqrt(v + eps) * g + b


def _geluk(x):
    return 0.5 * x * (1.0 + lax.erf(x * 0.7071067811865476))


def _dot(a, b):
    return jnp.dot(a, b, preferred_element_type=f32)


def _nspec(a):
    if a.ndim == 2 and a.shape[0] == NP_:
        w = a.shape[1]
        return pl.BlockSpec((BS, w), lambda i: (i, 0))
    if a.ndim == 3 and a.shape[1] == NP_:
        d0, _, w = a.shape
        return pl.BlockSpec((d0, BS, w), lambda i: (0, i, 0))
    nd = a.ndim
    return pl.BlockSpec(a.shape, lambda i: (0,) * nd)


def _tc_call(body, args, out_widths):
    outs = tuple(jax.ShapeDtypeStruct((NP_, w), f32) for w in out_widths)
    return pl.pallas_call(
        body,
        grid=(GRID,),
        in_specs=[_nspec(a) for a in args],
        out_specs=tuple(pl.BlockSpec((BS, w), lambda i: (i, 0)) for w in out_widths),
        out_shape=outs,
    )(*args)


def _t1_body(xp_r, xa_r, png, pnb, ang, anb,
             plW1, plb1, plg, plbln, plW2, plb2,
             alW1, alb1, alg, albln, alW2, alb2,
             g1W, g1as, g1ad,
             xp0_o, xa0_o, h1_o, acat_o, bcat_o):
    xp = _lnk(xp_r[...], png[...], pnb[...])
    xa = _lnk(xa_r[...], ang[...], anb[...])
    t = _geluk(_dot(xp, plW1[...]) + plb1[...])
    t = _lnk(t, plg[...], plbln[...])
    xp0 = _dot(t, plW2[...]) + plb2[...]
    t = _geluk(_dot(xa, alW1[...]) + alb1[...])
    t = _lnk(t, alg[...], albln[...])
    xa0 = _dot(t, alW2[...]) + alb2[...]
    h1 = _dot(xp0, g1W[...])
    hh = h1.reshape(BS, HEADS, DH)
    a_s = jnp.sum(hh * g1as[...], axis=-1)
    a_d = jnp.sum(hh * g1ad[...], axis=-1)
    xp0_o[...] = xp0
    xa0_o[...] = xa0
    h1_o[...] = h1
    acat_o[...] = jnp.concatenate([a_s, a_d], axis=-1)
    bcat_o[...] = jnp.concatenate([a_d, a_s], axis=-1)


def _t2_body(xp0_r, xa0_r, po, pa, ps, pc,
             g1b, n1g, n1b, sWl, sbl, sWr, n2g, n2b, gcnW,
             xp1_o, xp2_o, hs_o):
    o = po[...]
    pa_v = pa[...]
    asum = pa_v[:, :HEADS]
    att = (o.reshape(BS, HEADS, DH) / (asum[:, :, None] + 1e-16)).reshape(BS, D)
    att1 = _geluk(_lnk(att + g1b[...], n1g[...], n1b[...]))
    xp1 = att1 + xp0_r[...]
    ssum = ps[...]
    cnt = pc[...][:, 0:1]
    mean = ssum / jnp.maximum(cnt, 1.0)
    sage = _dot(mean, sWl[...]) + sbl[...] + _dot(xa0_r[...], sWr[...])
    a2p = _geluk(_lnk(sage, n2g[...], n2b[...]))
    xp2 = xp1 + 0.5 * a2p
    deg = pa_v[:, HEADS:HEADS + 1]
    dinv = jnp.where(deg > 0.0, lax.rsqrt(jnp.maximum(deg, 1e-30)), 0.0)
    hs = _dot(xp2, gcnW[...]) * dinv
    xp1_o[...] = xp1
    xp2_o[...] = xp2
    hs_o[...] = hs


def _t3_body(xp2_r, pg, pa, gcnb, g2W, g2as, g2ad,
             xp3_o, h2_o, acat_o, bcat_o):
    g = pg[...]
    deg = pa[...][:, HEADS:HEADS + 1]
    dinv = jnp.where(deg > 0.0, lax.rsqrt(jnp.maximum(deg, 1e-30)), 0.0)
    conv = _geluk(g * dinv + gcnb[...])
    xp3 = xp2_r[...] + 0.3 * conv
    h2 = _dot(xp3, g2W[...])
    hh = h2.reshape(BS, HEADS, DH)
    a_s = jnp.sum(hh * g2as[...], axis=-1)
    a_d = jnp.sum(hh * g2ad[...], axis=-1)
    xp3_o[...] = xp3
    h2_o[...] = h2
    acat_o[...] = jnp.concatenate([a_s, a_d], axis=-1)
    bcat_o[...] = jnp.concatenate([a_d, a_s], axis=-1)


def _t4_body(xp0_r, xp1_r, xp3_r, po, pa, g2b, n4g, n4b, sw,
             cW1, cb1, cg, cbln, cW2, cb2, cW3, cb3,
             out_o):
    o = po[...]
    asum = pa[...][:, :HEADS]
    att = (o.reshape(BS, HEADS, DH) / (asum[:, :, None] + 1e-16)).reshape(BS, D)
    att2 = _geluk(_lnk(att + g2b[...], n4g[...], n4b[...]))
    xp4 = xp3_r[...] + att2
    swv = sw[...]
    ms = swv[0, 0] * xp0_r[...] + swv[0, 1] * xp1_r[...] + swv[0, 2] * xp4
    comb = jnp.concatenate([ms, xp4], axis=-1)
    h1c = _geluk(_lnk(_dot(comb, cW1[...]) + cb1[...], cg[...], cbln[...]))
    h2c = _geluk(_dot(h1c, cW2[...]) + cb2[...])
    out_o[...] = _dot(h2c, cW3[...]) + cb3[...]


def _pad_edges(src, dst, epad):
    npad = epad - src.shape[0]
    src = jnp.concatenate([src, jnp.zeros((npad,), jnp.int32)])
    dst = jnp.concatenate([dst, jnp.full((npad,), DUMMY, jnp.int32)])
    return src, dst


def kernel(x_patent, x_author, edge_index_pp, edge_index_ap, edge_index_pa,
           params):
    P = params
    r1 = lambda p: p.reshape(1, -1)

    xp_in = jnp.pad(x_patent, ((0, NP_ - N), (0, 0)))
    xa_in = jnp.pad(x_author, ((0, NP_ - N), (0, 0)))

    loops = jnp.arange(N, dtype=jnp.int32)
    pp_src = jnp.concatenate([edge_index_pp[0], loops])
    pp_dst = jnp.concatenate([edge_index_pp[1], loops])
    pp_src, pp_dst = _pad_edges(pp_src, pp_dst, EPAD_SL)
    ap_src, ap_dst = _pad_edges(edge_index_ap[0], edge_index_ap[1], EPAD_PLAIN)

    zeros128 = jnp.zeros((RPS, D), f32)
    ones128 = jnp.ones((CH, D), f32)
    padw = ((0, 0), (0, D - 16))

    sw = jax.nn.softmax(P['scale_w'])
    sw_arr = jnp.zeros((1, D), f32).at[0, :3].set(sw)
    cW3 = jnp.pad(P['c_W3'], ((0, 0), (0, D - NCLS)))
    cb3 = jnp.pad(P['c_b3'], ((0, D - NCLS))).reshape(1, D)

    xp0, xa0, h1, acat1, bcat1 = _tc_call(
        _t1_body,
        [xp_in, xa_in, r1(P['pn_g']), r1(P['pn_b']), r1(P['an_g']), r1(P['an_b']),
         P['pl_W1'], r1(P['pl_b1']), r1(P['pl_g']), r1(P['pl_bln']), P['pl_W2'], r1(P['pl_b2']),
         P['al_W1'], r1(P['al_b1']), r1(P['al_g']), r1(P['al_bln']), P['al_W2'], r1(P['al_b2']),
         P['g1_W'], P['g1_as'], P['g1_ad']],
        (D, D, D, 16, 16))

    g1 = _GAT_G(h1, jnp.pad(acat1, padw), jnp.pad(bcat1, padw),
                pp_src, pp_dst, zeros128)
    g1_out, g1_aux = g1[0], g1[1]
    sage = _PLAIN_P(xa0, ap_src, ap_dst, zeros128, ones128)
    sage_sum, sage_cnt = sage[0], sage[1]

    xp1, xp2, hs = _tc_call(
        _t2_body,
        [xp0, xa0, g1_out, g1_aux, sage_sum, sage_cnt,
         r1(P['g1_b']), r1(P['n1_g']), r1(P['n1_b']),
         P['sap_Wl'], r1(P['sap_bl']), P['sap_Wr'],
         r1(P['n2_g']), r1(P['n2_b']), P['gcn_W']],
        (D, D, D))

    gcn = _PLAIN_G(hs, pp_src, pp_dst, zeros128, ones128)

    xp3, h2, acat2, bcat2 = _tc_call(
        _t3_body,
        [xp2, gcn[0], g1_aux, r1(P['gcn_b']), P['g2_W'], P['g2_as'], P['g2_ad']],
        (D, D, 16, 16))

    g2 = _GAT_G(h2, jnp.pad(acat2, padw), jnp.pad(bcat2, padw),
                pp_src, pp_dst, zeros128)

    out_pad, = _tc_call(
        _t4_body,
        [xp0, xp1, xp3, g2[0], g2[1],
         r1(P['g2_b']), r1(P['n4_g']), r1(P['n4_b']), sw_arr,
         P['c_W1'], r1(P['c_b1']), r1(P['c_g']), r1(P['c_bln']),
         P['c_W2'], r1(P['c_b2']), cW3, cb3],
        (D,))

    return out_pad[:N, :NCLS]


# speedup vs baseline: 29.9933x; 1.1054x over previous
import functools

import jax
import jax.numpy as jnp
from jax import lax
from jax.experimental import pallas as pl
from jax.experimental.pallas import tpu as pltpu
from jax.experimental.pallas import tpu_sc as plsc

N = 10000
NP_ = 10112
D = 128
HEADS = 8
DH = 16
NCLS = 40
NC, NS = 2, 16
CH = 80
RPS = NP_ // NS
DUMMY = 10048
BS = 2528
GRID = NP_ // BS

E_RAW = 320000
E_SL = E_RAW + N
NCHS_PLAIN = 250
NCHS_GAT = 258
NCHW_GCN = NCHS_GAT // 2
EPAD_PLAIN = NS * CH * NCHS_PLAIN
EPAD_SL = NS * CH * NCHS_GAT
f32 = jnp.float32


_MESH = plsc.VectorSubcoreMesh(core_axis_name="c", subcore_axis_name="s",
                               num_cores=NC, num_subcores=NS)


def _make_plain(nchunks):
    @functools.partial(
        pl.kernel,
        out_type=jax.ShapeDtypeStruct((NC, NP_, D), f32),
        mesh=_MESH,
        scratch_types=[
            pltpu.VMEM_SHARED((NP_, D), f32),
            pltpu.VMEM((CH,), jnp.int32),
            pltpu.VMEM((CH,), jnp.int32),
            pltpu.VMEM((CH, D), f32),
            pltpu.VMEM((CH, D), f32),
            pltpu.SemaphoreType.DMA,
        ],
    )
    def plain_pass(table, srci, dsti, zeros128, ones128,
                   out128,
                   acc128, idx_s, idx_d, rows, ones_v, sem):
        c = lax.axis_index("c")
        s = lax.axis_index("s")
        r0 = s * RPS
        pltpu.sync_copy(zeros128, acc128.at[pl.ds(r0, RPS)])
        pltpu.sync_copy(ones128, ones_v)
        plsc.subcore_barrier()

        base0 = s * (nchunks * CH)

        @pl.loop(0, nchunks)
        def _(i):
            b = base0 + i * CH
            pltpu.sync_copy(dsti.at[pl.ds(b, CH)], idx_d)

            @pl.when(c == 0)
            def _():
                pltpu.sync_copy(srci.at[pl.ds(b, CH)], idx_s)
                pltpu.async_copy(table.at[idx_s], rows, sem).wait()
                pltpu.sync_copy(rows, acc128.at[idx_d], add=True)

            @pl.when(c == 1)
            def _():
                pltpu.sync_copy(ones_v, acc128.at[idx_d], add=True)

        plsc.subcore_barrier()
        pltpu.sync_copy(acc128.at[pl.ds(r0, RPS)], out128.at[c, pl.ds(r0, RPS)])

    return plain_pass


def _make_plain_split(nchunks):
    @functools.partial(
        pl.kernel,
        out_type=jax.ShapeDtypeStruct((NC, NP_, D), f32),
        mesh=_MESH,
        scratch_types=[
            pltpu.VMEM_SHARED((NP_, D), f32),
            pltpu.VMEM((CH,), jnp.int32),
            pltpu.VMEM((CH,), jnp.int32),
            pltpu.VMEM((CH, D), f32),
            pltpu.SemaphoreType.DMA,
        ],
    )
    def plain_split(table, srci, dsti, zeros128,
                    out128,
                    acc128, idx_s, idx_d, rows, sem):
        c = lax.axis_index("c")
        s = lax.axis_index("s")
        w = c * NS + s
        r0 = s * RPS
        pltpu.sync_copy(zeros128, acc128.at[pl.ds(r0, RPS)])
        plsc.subcore_barrier()

        base0 = w * (nchunks * CH)

        @pl.loop(0, nchunks)
        def _(i):
            b = base0 + i * CH
            pltpu.sync_copy(srci.at[pl.ds(b, CH)], idx_s)
            pltpu.sync_copy(dsti.at[pl.ds(b, CH)], idx_d)
            pltpu.async_copy(table.at[idx_s], rows, sem).wait()
            pltpu.sync_copy(rows, acc128.at[idx_d], add=True)

        plsc.subcore_barrier()
        pltpu.sync_copy(acc128.at[pl.ds(r0, RPS)], out128.at[c, pl.ds(r0, RPS)])

    return plain_split


def _make_gat(nchunks):
    @functools.partial(
        pl.kernel,
        out_type=jax.ShapeDtypeStruct((NC, NP_, D), f32),
        mesh=_MESH,
        scratch_types=[
            pltpu.VMEM_SHARED((NP_, D), f32),
            pltpu.VMEM((CH,), jnp.int32),
            pltpu.VMEM((CH,), jnp.int32),
            pltpu.VMEM((CH, D), f32),
            pltpu.VMEM((CH, D), f32),
            pltpu.VMEM((CH, D), f32),
            pltpu.VMEM((CH, D), f32),
            pltpu.SemaphoreType.DMA,
        ],
    )
    def gat_pass(htab, acat, bcat, srci, dsti, zeros128,
                 out128,
                 acc128, idx_s, idx_d, hbuf, abuf, bbuf, wbuf, sem):
        c = lax.axis_index("c")
        s = lax.axis_index("s")
        r0 = s * RPS
        pltpu.sync_copy(zeros128, acc128.at[pl.ds(r0, RPS)])

        zero16 = jnp.zeros((16,), f32)

        @pl.when(c == 1)
        def _():
            @pl.loop(0, CH)
            def _(e):
                for k in range(1, HEADS):
                    wbuf[e, pl.ds(k * DH, DH)] = zero16

        plsc.subcore_barrier()

        base0 = s * (nchunks * CH)
        lanes = lax.iota(jnp.int32, 16)

        @pl.loop(0, nchunks)
        def _(i):
            b = base0 + i * CH
            pltpu.sync_copy(srci.at[pl.ds(b, CH)], idx_s)
            pltpu.sync_copy(dsti.at[pl.ds(b, CH)], idx_d)

            @pl.when(c == 0)
            def _():
                ca = pltpu.async_copy(acat.at[idx_s], abuf, sem)
                cb = pltpu.async_copy(bcat.at[idx_d], bbuf, sem)
                chh = pltpu.async_copy(htab.at[idx_s], hbuf, sem)
                ca.wait()
                cb.wait()
                chh.wait()

                @pl.loop(0, CH)
                def _(e):
                    v = abuf[e, pl.ds(0, 16)] + bbuf[e, pl.ds(0, 16)]
                    t = jnp.exp(jnp.where(v >= 0.0, v, 0.2 * v))
                    for j in range(HEADS):
                        hv = hbuf[e, pl.ds(j * DH, DH)]
                        wbuf[e, pl.ds(j * DH, DH)] = hv * t[j]

            @pl.when(c == 1)
            def _():
                ca = pltpu.async_copy(acat.at[idx_s], abuf, sem)
                cb = pltpu.async_copy(bcat.at[idx_d], bbuf, sem)
                ca.wait()
                cb.wait()

                @pl.loop(0, CH)
                def _(e):
                    v = abuf[e, pl.ds(0, 16)] + bbuf[e, pl.ds(0, 16)]
                    vc = jnp.where(lanes < 8,
                                   jnp.where(v >= 0.0, v, 0.2 * v), 0.0)
                    wbuf[e, pl.ds(0, 16)] = jnp.exp(vc)

            pltpu.sync_copy(wbuf, acc128.at[idx_d], add=True)

        plsc.subcore_barrier()
        pltpu.sync_copy(acc128.at[pl.ds(r0, RPS)], out128.at[c, pl.ds(r0, RPS)])

    return gat_pass


_PLAIN_P = _make_plain(NCHS_PLAIN)
_PLAIN_S = _make_plain_split(NCHW_GCN)
_GAT_G = _make_gat(NCHS_GAT)


def _lnk(x, g, b, eps=1e-5):
    m = jnp.mean(x, axis=-1, keepdims=True)
    v = jnp.mean((x - m) * (x - m), axis=-1, keepdims=True)
    return (x - m) * lax.rsqrt(v + eps) * g + b


def _geluk(x):
    return 0.5 * x * (1.0 + lax.erf(x * 0.7071067811865476))


def _dot(a, b):
    return jnp.dot(a, b, preferred_element_type=f32)


def _nspec(a):
    if a.ndim == 2 and a.shape[0] == NP_:
        w = a.shape[1]
        return pl.BlockSpec((BS, w), lambda i: (i, 0))
    if a.ndim == 3 and a.shape[1] == NP_:
        d0, _, w = a.shape
        return pl.BlockSpec((d0, BS, w), lambda i: (0, i, 0))
    nd = a.ndim
    return pl.BlockSpec(a.shape, lambda i: (0,) * nd)


def _tc_call(body, args, out_widths):
    outs = tuple(jax.ShapeDtypeStruct((NP_, w), f32) for w in out_widths)
    return pl.pallas_call(
        body,
        grid=(GRID,),
        in_specs=[_nspec(a) for a in args],
        out_specs=tuple(pl.BlockSpec((BS, w), lambda i: (i, 0)) for w in out_widths),
        out_shape=outs,
    )(*args)


def _t1_body(xp_r, xa_r, png, pnb, ang, anb,
             plW1, plb1, plg, plbln, plW2, plb2,
             alW1, alb1, alg, albln, alW2, alb2,
             g1W, g1as, g1ad,
             xp0_o, xa0_o, h1_o, acat_o, bcat_o):
    xp = _lnk(xp_r[...], png[...], pnb[...])
    xa = _lnk(xa_r[...], ang[...], anb[...])
    t = _geluk(_dot(xp, plW1[...]) + plb1[...])
    t = _lnk(t, plg[...], plbln[...])
    xp0 = _dot(t, plW2[...]) + plb2[...]
    t = _geluk(_dot(xa, alW1[...]) + alb1[...])
    t = _lnk(t, alg[...], albln[...])
    xa0 = _dot(t, alW2[...]) + alb2[...]
    h1 = _dot(xp0, g1W[...])
    hh = h1.reshape(BS, HEADS, DH)
    a_s = jnp.sum(hh * g1as[...], axis=-1)
    a_d = jnp.sum(hh * g1ad[...], axis=-1)
    xp0_o[...] = xp0
    xa0_o[...] = xa0
    h1_o[...] = h1
    acat_o[...] = jnp.concatenate([a_s, a_d], axis=-1)
    bcat_o[...] = jnp.concatenate([a_d, a_s], axis=-1)


def _t2_body(xp0_r, xa0_r, po, pa, ps, pc,
             g1b, n1g, n1b, sWl, sbl, sWr, n2g, n2b, gcnW,
             xp1_o, xp2_o, hs_o):
    o = po[...]
    pa_v = pa[...]
    asum = pa_v[:, :HEADS]
    att = (o.reshape(BS, HEADS, DH) / (asum[:, :, None] + 1e-16)).reshape(BS, D)
    att1 = _geluk(_lnk(att + g1b[...], n1g[...], n1b[...]))
    xp1 = att1 + xp0_r[...]
    ssum = ps[...]
    cnt = pc[...][:, 0:1]
    mean = ssum / jnp.maximum(cnt, 1.0)
    sage = _dot(mean, sWl[...]) + sbl[...] + _dot(xa0_r[...], sWr[...])
    a2p = _geluk(_lnk(sage, n2g[...], n2b[...]))
    xp2 = xp1 + 0.5 * a2p
    deg = pa_v[:, HEADS:HEADS + 1]
    dinv = jnp.where(deg > 0.0, lax.rsqrt(jnp.maximum(deg, 1e-30)), 0.0)
    hs = _dot(xp2, gcnW[...]) * dinv
    xp1_o[...] = xp1
    xp2_o[...] = xp2
    hs_o[...] = hs


def _t3_body(xp2_r, pg, pa, gcnb, g2W, g2as, g2ad,
             xp3_o, h2_o, acat_o, bcat_o):
    g = pg[0] + pg[1]
    deg = pa[...][:, HEADS:HEADS + 1]
    dinv = jnp.where(deg > 0.0, lax.rsqrt(jnp.maximum(deg, 1e-30)), 0.0)
    conv = _geluk(g * dinv + gcnb[...])
    xp3 = xp2_r[...] + 0.3 * conv
    h2 = _dot(xp3, g2W[...])
    hh = h2.reshape(BS, HEADS, DH)
    a_s = jnp.sum(hh * g2as[...], axis=-1)
    a_d = jnp.sum(hh * g2ad[...], axis=-1)
    xp3_o[...] = xp3
    h2_o[...] = h2
    acat_o[...] = jnp.concatenate([a_s, a_d], axis=-1)
    bcat_o[...] = jnp.concatenate([a_d, a_s], axis=-1)


def _t4_body(xp0_r, xp1_r, xp3_r, po, pa, g2b, n4g, n4b, sw,
             cW1, cb1, cg, cbln, cW2, cb2, cW3, cb3,
             out_o):
    o = po[...]
    asum = pa[...][:, :HEADS]
    att = (o.reshape(BS, HEADS, DH) / (asum[:, :, None] + 1e-16)).reshape(BS, D)
    att2 = _geluk(_lnk(att + g2b[...], n4g[...], n4b[...]))
    xp4 = xp3_r[...] + att2
    swv = sw[...]
    ms = swv[0, 0] * xp0_r[...] + swv[0, 1] * xp1_r[...] + swv[0, 2] * xp4
    comb = jnp.concatenate([ms, xp4], axis=-1)
    h1c = _geluk(_lnk(_dot(comb, cW1[...]) + cb1[...], cg[...], cbln[...]))
    h2c = _geluk(_dot(h1c, cW2[...]) + cb2[...])
    out_o[...] = _dot(h2c, cW3[...]) + cb3[...]


def _pad_edges(src, dst, epad):
    npad = epad - src.shape[0]
    src = jnp.concatenate([src, jnp.zeros((npad,), jnp.int32)])
    dst = jnp.concatenate([dst, jnp.full((npad,), DUMMY, jnp.int32)])
    return src, dst


def kernel(x_patent, x_author, edge_index_pp, edge_index_ap, edge_index_pa,
           params):
    P = params
    r1 = lambda p: p.reshape(1, -1)

    xp_in = jnp.pad(x_patent, ((0, NP_ - N), (0, 0)))
    xa_in = jnp.pad(x_author, ((0, NP_ - N), (0, 0)))

    loops = jnp.arange(N, dtype=jnp.int32)
    pp_src = jnp.concatenate([edge_index_pp[0], loops])
    pp_dst = jnp.concatenate([edge_index_pp[1], loops])
    pp_src, pp_dst = _pad_edges(pp_src, pp_dst, EPAD_SL)
    ap_src, ap_dst = _pad_edges(edge_index_ap[0], edge_index_ap[1], EPAD_PLAIN)

    zeros128 = jnp.zeros((RPS, D), f32)
    ones128 = jnp.ones((CH, D), f32)
    padw = ((0, 0), (0, D - 16))

    sw = jax.nn.softmax(P['scale_w'])
    sw_arr = jnp.zeros((1, D), f32).at[0, :3].set(sw)
    cW3 = jnp.pad(P['c_W3'], ((0, 0), (0, D - NCLS)))
    cb3 = jnp.pad(P['c_b3'], ((0, D - NCLS))).reshape(1, D)

    xp0, xa0, h1, acat1, bcat1 = _tc_call(
        _t1_body,
        [xp_in, xa_in, r1(P['pn_g']), r1(P['pn_b']), r1(P['an_g']), r1(P['an_b']),
         P['pl_W1'], r1(P['pl_b1']), r1(P['pl_g']), r1(P['pl_bln']), P['pl_W2'], r1(P['pl_b2']),
         P['al_W1'], r1(P['al_b1']), r1(P['al_g']), r1(P['al_bln']), P['al_W2'], r1(P['al_b2']),
         P['g1_W'], P['g1_as'], P['g1_ad']],
        (D, D, D, 16, 16))

    g1 = _GAT_G(h1, jnp.pad(acat1, padw), jnp.pad(bcat1, padw),
                pp_src, pp_dst, zeros128)
    g1_out, g1_aux = g1[0], g1[1]
    sage = _PLAIN_P(xa0, ap_src, ap_dst, zeros128, ones128)
    sage_sum, sage_cnt = sage[0], sage[1]

    xp1, xp2, hs = _tc_call(
        _t2_body,
        [xp0, xa0, g1_out, g1_aux, sage_sum, sage_cnt,
         r1(P['g1_b']), r1(P['n1_g']), r1(P['n1_b']),
         P['sap_Wl'], r1(P['sap_bl']), P['sap_Wr'],
         r1(P['n2_g']), r1(P['n2_b']), P['gcn_W']],
        (D, D, D))

    gcn = _PLAIN_S(hs, pp_src, pp_dst, zeros128)

    xp3, h2, acat2, bcat2 = _tc_call(
        _t3_body,
        [xp2, gcn, g1_aux, r1(P['gcn_b']), P['g2_W'], P['g2_as'], P['g2_ad']],
        (D, D, 16, 16))

    g2 = _GAT_G(h2, jnp.pad(acat2, padw), jnp.pad(bcat2, padw),
                pp_src, pp_dst, zeros128)

    out_pad, = _tc_call(
        _t4_body,
        [xp0, xp1, xp3, g2[0], g2[1],
         r1(P['g2_b']), r1(P['n4_g']), r1(P['n4_b']), sw_arr,
         P['c_W1'], r1(P['c_b1']), r1(P['c_g']), r1(P['c_bln']),
         P['c_W2'], r1(P['c_b2']), cW3, cb3],
        (D,))

    return out_pad[:N, :NCLS]
